# baseline (speedup 1.0000x reference)
import functools

import jax
import jax.numpy as jnp
from jax import lax
from jax.experimental import pallas as pl
from jax.experimental.pallas import tpu as pltpu
from jax.experimental.pallas import tpu_sc as plsc

BATCH = 16384
EMB = 32
HID = 64
NROW = 1000000

NC = 2
NS = 16
NW = NC * NS

NT_FULL = 7812
TAIL_START = NT_FULL * 128
CROWS = 640
NCH = -(-245 * 128 // CROWS)
STAGE = 256
DUMMY = BATCH
NV = BATCH // 16


def _compact(refs, offset, values, m, trash):
  cum = jnp.cumsum(m.astype(jnp.int32))
  pos = jnp.where(m, offset + cum - 1, trash)
  for ref, val in zip(refs, values):
    plsc.store_scatter(ref, [pos], val)
  pc = cum[15]
  return offset + pc, pc


def _one_table(idx_hbm, tab_hbm, out_hbm, idx_v, my_idx, my_j, chunk_v,
               st_rows, st_j, st_jx, st_pos, sem_i, sem_c, sem_o, rs, re):
  iota = lax.iota(jnp.int32, 16)

  pltpu.async_copy(idx_hbm, idx_v, sem_i).wait()

  def scan_ids(v, cnt):
    ids = idx_v[pl.ds(v * 16, 16)]
    m = (ids >= rs) & (ids < re)

    def hit(c):
      c2, _ = _compact([my_idx, my_j], c, [ids, iota + v * 16], m, BATCH + 8)
      return c2

    return lax.cond(jnp.any(m), hit, lambda c: c, cnt)

  cnt = lax.fori_loop(0, NV, scan_ids, 0)
  nv_mine = (cnt + 15) // 16

  dummy_vec = jnp.full((16,), DUMMY, jnp.int32)
  for t in range(STAGE // 16):
    st_j[pl.ds(t * 16, 16)] = dummy_vec

  def chunk_start(c):
    return pl.multiple_of(
        jnp.minimum(rs + c * CROWS, re - CROWS), 128)

  def copy_chunk(c):
    cs = chunk_start(c)
    return pltpu.make_async_copy(
        tab_hbm.at[:, pl.ds(cs, CROWS)], chunk_v.at[c % 2], sem_c)

  def flush(wcnt):
    for t in range(STAGE // 16):
      st_jx[pl.ds(t * 16, 16)] = st_j[pl.ds(t * 16, 16)]
    pltpu.async_copy(st_rows, out_hbm.at[st_jx], sem_o).wait()
    return 0 * wcnt

  copy_chunk(0).start()

  def do_chunk(c, wcnt):
    @pl.when(c + 1 < NCH)
    def _prefetch():
      copy_chunk(c + 1).start()

    copy_chunk(c).wait()
    cs = chunk_start(c)
    par = c % 2

    def scan_mine(v, wc):
      ids = my_idx[pl.ds(v * 16, 16)]
      jv = my_j[pl.ds(v * 16, 16)]
      m = (ids >= cs) & (ids < cs + CROWS) & (iota + v * 16 < cnt)

      def hit(wc):
        wc = lax.cond(wc > STAGE - 16, flush, lambda w: w, wc)
        _, pc = _compact([st_j, st_pos], wc, [jv, ids - cs], m, STAGE + 8)

        def one_entry(e, _):
          pos = st_pos[pl.ds(wc + e, 16)][0]
          psplat = jnp.full((16,), pos, jnp.int32)
          par_splat = jnp.full((16,), par, jnp.int32)
          row_splat = jnp.full((16,), wc + e, jnp.int32)
          g_lo = plsc.load_gather(chunk_v, [par_splat, iota, psplat])
          g_hi = plsc.load_gather(chunk_v, [par_splat, iota + 16, psplat])
          plsc.store_scatter(st_rows, [row_splat, iota], g_lo)
          plsc.store_scatter(st_rows, [row_splat, iota + 16], g_hi)
          return _

        lax.fori_loop(0, pc, one_entry, 0)
        return wc + pc

      return lax.cond(jnp.any(m), hit, lambda w: w, wc)

    return lax.fori_loop(0, nv_mine, scan_mine, wcnt)

  wcnt_final = lax.fori_loop(0, NCH, do_chunk, 0)
  flush(wcnt_final)


def _sc_body(uid_hbm, iid_hbm, utab_hbm, itab_hbm, out_u_hbm, out_i_hbm,
             idx_v, my_idx, my_j, chunk_v, st_rows, st_j, st_jx, st_pos,
             sem_i, sem_c, sem_o):
  wid = lax.axis_index("s") * NC + lax.axis_index("c")
  ts = (NT_FULL * wid) // NW
  te = (NT_FULL * (wid + 1)) // NW
  rs = pl.multiple_of(ts * 128, 128)
  re = pl.multiple_of(te * 128, 128)
  _one_table(uid_hbm, utab_hbm, out_u_hbm, idx_v, my_idx, my_j, chunk_v,
             st_rows, st_j, st_jx, st_pos, sem_i, sem_c, sem_o, rs, re)
  _one_table(iid_hbm, itab_hbm, out_i_hbm, idx_v, my_idx, my_j, chunk_v,
             st_rows, st_j, st_jx, st_pos, sem_i, sem_c, sem_o, rs, re)


@functools.cache
def _sc_gather():
  return pl.kernel(
      _sc_body,
      out_type=(jax.ShapeDtypeStruct((BATCH + 1, 128), jnp.float32),
                jax.ShapeDtypeStruct((BATCH + 1, 128), jnp.float32)),
      mesh=plsc.VectorSubcoreMesh(
          core_axis_name="c", subcore_axis_name="s",
          num_cores=NC, num_subcores=NS),
      scratch_types=[
          pltpu.VMEM((BATCH,), jnp.int32),
          pltpu.VMEM((BATCH + 16,), jnp.int32),
          pltpu.VMEM((BATCH + 16,), jnp.int32),
          pltpu.VMEM((2, EMB, CROWS), jnp.float32),
          pltpu.VMEM((STAGE, 128), jnp.float32),
          pltpu.VMEM((STAGE + 16,), jnp.int32),
          pltpu.VMEM((STAGE,), jnp.int32),
          pltpu.VMEM((STAGE + 16,), jnp.int32),
          pltpu.SemaphoreType.DMA,
          pltpu.SemaphoreType.DMA,
          pltpu.SemaphoreType.DMA,
      ],
      compiler_params=pltpu.CompilerParams(use_tc_tiling_on_sc=True, needs_layout_passes=False),
  )


def _mlp_body(u_ref, i_ref, uid_ref, iid_ref, tu_ref, ti_ref, w1u_ref,
              w1i_ref, b1_ref, w2_ref, b2_ref, wout_ref, bout_ref, out_ref):
  cols = lax.broadcasted_iota(jnp.int32, (1, 64), 1)
  uid = uid_ref[...]
  iid = iid_ref[...]
  oh_u = (uid - TAIL_START == cols).astype(jnp.float32)
  oh_i = (iid - TAIL_START == cols).astype(jnp.float32)
  u = jnp.where(uid >= TAIL_START, oh_u @ tu_ref[...], u_ref[:, :EMB])
  i = jnp.where(iid >= TAIL_START, oh_i @ ti_ref[...], i_ref[:, :EMB])
  h = u @ w1u_ref[...] + i @ w1i_ref[...] + b1_ref[...]
  h = jnp.maximum(h, 0.0)
  h = jnp.maximum(h @ w2_ref[...] + b2_ref[...], 0.0)
  y = jnp.sum(h * wout_ref[...], axis=1, keepdims=True) + bout_ref[...]
  out_ref[...] = y


def _mlp_call(gu, gi, uid2, iid2, tail_u, tail_i, w1u, w1i, b1, w2, b2,
              wout_t, bout, block_b):
  grid = (BATCH // block_b,)
  full = lambda shape: pl.BlockSpec(shape, lambda b: (0,) * len(shape))
  return pl.pallas_call(
      _mlp_body,
      grid=grid,
      in_specs=[
          pl.BlockSpec((block_b, 128), lambda b: (b, 0)),
          pl.BlockSpec((block_b, 128), lambda b: (b, 0)),
          pl.BlockSpec((block_b, 1), lambda b: (b, 0)),
          pl.BlockSpec((block_b, 1), lambda b: (b, 0)),
          full((64, EMB)),
          full((64, EMB)),
          full((EMB, HID)),
          full((EMB, HID)),
          full((1, HID)),
          full((HID, HID)),
          full((1, HID)),
          full((1, HID)),
          full((1, 1)),
      ],
      out_specs=pl.BlockSpec((block_b, 1), lambda b: (b, 0)),
      out_shape=jax.ShapeDtypeStruct((BATCH, 1), jnp.float32),
  )(gu, gi, uid2, iid2, tail_u, tail_i, w1u, w1i, b1, w2, b2, wout_t, bout)


def kernel(userID, ItemID, user_table, item_table, W1, b1, W2, b2, Wout, bout):
  gu_p, gi_p = _sc_gather()(userID, ItemID, user_table.T, item_table.T)
  gu = gu_p[:BATCH]
  gi = gi_p[:BATCH]
  tail_u = user_table[TAIL_START:]
  tail_i = item_table[TAIL_START:]
  y = _mlp_call(gu, gi, userID.reshape(BATCH, 1), ItemID.reshape(BATCH, 1),
                tail_u, tail_i, W1[:EMB], W1[EMB:], b1.reshape(1, HID),
                W2, b2.reshape(1, HID), Wout.reshape(1, HID),
                bout.reshape(1, 1), block_b=2048)
  return y[:, 0]


# speedup vs baseline: 1.0356x; 1.0356x over previous
import functools

import jax
import jax.numpy as jnp
from jax import lax
from jax.experimental import pallas as pl
from jax.experimental.pallas import tpu as pltpu
from jax.experimental.pallas import tpu_sc as plsc

BATCH = 16384
EMB = 32
HID = 64
NROW = 1000000

NC = 2
NS = 16
NW = NC * NS

NT_FULL = 7812
TAIL_START = NT_FULL * 128
CROWS = 640
NCH = -(-245 * 128 // CROWS)
STAGE = 256
DUMMY = BATCH
NV = BATCH // 16


def _compact(refs, offset, values, m, trash):
  cum = jnp.cumsum(m.astype(jnp.int32))
  pos = jnp.where(m, offset + cum - 1, trash)
  for ref, val in zip(refs, values):
    plsc.store_scatter(ref, [pos], val)
  pc = cum[15]
  return offset + pc, pc


def _one_table(idx_hbm, tab_hbm, out_hbm, idx_v, my_idx, my_j, chunk_v,
               st_rows, st_j, st_jx, st_pos, sem_i, sem_c, sem_o, rs, re):
  iota = lax.iota(jnp.int32, 16)

  pltpu.async_copy(idx_hbm, idx_v, sem_i).wait()

  def scan_ids(v, cnt):
    ids = idx_v[pl.ds(v * 16, 16)]
    m = (ids >= rs) & (ids < re)

    def hit(c):
      c2, _ = _compact([my_idx, my_j], c, [ids, iota + v * 16], m, BATCH + 8)
      return c2

    return lax.cond(jnp.any(m), hit, lambda c: c, cnt)

  cnt = lax.fori_loop(0, NV, scan_ids, 0, unroll=4)
  nv_mine = (cnt + 15) // 16

  dummy_vec = jnp.full((16,), DUMMY, jnp.int32)
  for t in range(STAGE // 16):
    st_j[pl.ds(t * 16, 16)] = dummy_vec

  def chunk_start(c):
    return pl.multiple_of(
        jnp.minimum(rs + c * CROWS, re - CROWS), 128)

  def copy_chunk(c):
    cs = chunk_start(c)
    return pltpu.make_async_copy(
        tab_hbm.at[:, pl.ds(cs, CROWS)], chunk_v.at[c % 2], sem_c)

  def flush(wcnt):
    for t in range(STAGE // 16):
      st_jx[pl.ds(t * 16, 16)] = st_j[pl.ds(t * 16, 16)]
    pltpu.async_copy(st_rows, out_hbm.at[st_jx], sem_o).wait()
    return 0 * wcnt

  copy_chunk(0).start()

  def do_chunk(c, wcnt):
    @pl.when(c + 1 < NCH)
    def _prefetch():
      copy_chunk(c + 1).start()

    copy_chunk(c).wait()
    cs = chunk_start(c)
    par = c % 2

    def scan_mine(v, wc):
      ids = my_idx[pl.ds(v * 16, 16)]
      jv = my_j[pl.ds(v * 16, 16)]
      m = (ids >= cs) & (ids < cs + CROWS) & (iota + v * 16 < cnt)

      def hit(wc):
        wc = lax.cond(wc > STAGE - 16, flush, lambda w: w, wc)
        _, pc = _compact([st_j, st_pos], wc, [jv, ids - cs], m, STAGE + 8)

        def one_entry(e, _):
          pos = st_pos[pl.ds(wc + e, 16)][0]
          psplat = jnp.full((16,), pos, jnp.int32)
          par_splat = jnp.full((16,), par, jnp.int32)
          row_splat = jnp.full((16,), wc + e, jnp.int32)
          g_lo = plsc.load_gather(chunk_v, [par_splat, iota, psplat])
          g_hi = plsc.load_gather(chunk_v, [par_splat, iota + 16, psplat])
          plsc.store_scatter(st_rows, [row_splat, iota], g_lo)
          plsc.store_scatter(st_rows, [row_splat, iota + 16], g_hi)
          return _

        lax.fori_loop(0, pc, one_entry, 0)
        return wc + pc

      return lax.cond(jnp.any(m), hit, lambda w: w, wc)

    return lax.fori_loop(0, nv_mine, scan_mine, wcnt)

  wcnt_final = lax.fori_loop(0, NCH, do_chunk, 0)
  flush(wcnt_final)


def _sc_body(uid_hbm, iid_hbm, utab_hbm, itab_hbm, out_u_hbm, out_i_hbm,
             idx_v, my_idx, my_j, chunk_v, st_rows, st_j, st_jx, st_pos,
             sem_i, sem_c, sem_o):
  wid = lax.axis_index("s") * NC + lax.axis_index("c")
  ts = (NT_FULL * wid) // NW
  te = (NT_FULL * (wid + 1)) // NW
  rs = pl.multiple_of(ts * 128, 128)
  re = pl.multiple_of(te * 128, 128)
  _one_table(uid_hbm, utab_hbm, out_u_hbm, idx_v, my_idx, my_j, chunk_v,
             st_rows, st_j, st_jx, st_pos, sem_i, sem_c, sem_o, rs, re)
  _one_table(iid_hbm, itab_hbm, out_i_hbm, idx_v, my_idx, my_j, chunk_v,
             st_rows, st_j, st_jx, st_pos, sem_i, sem_c, sem_o, rs, re)


@functools.cache
def _sc_gather():
  return pl.kernel(
      _sc_body,
      out_type=(jax.ShapeDtypeStruct((BATCH + 1, 128), jnp.float32),
                jax.ShapeDtypeStruct((BATCH + 1, 128), jnp.float32)),
      mesh=plsc.VectorSubcoreMesh(
          core_axis_name="c", subcore_axis_name="s",
          num_cores=NC, num_subcores=NS),
      scratch_types=[
          pltpu.VMEM((BATCH,), jnp.int32),
          pltpu.VMEM((BATCH + 16,), jnp.int32),
          pltpu.VMEM((BATCH + 16,), jnp.int32),
          pltpu.VMEM((2, EMB, CROWS), jnp.float32),
          pltpu.VMEM((STAGE, 128), jnp.float32),
          pltpu.VMEM((STAGE + 16,), jnp.int32),
          pltpu.VMEM((STAGE,), jnp.int32),
          pltpu.VMEM((STAGE + 16,), jnp.int32),
          pltpu.SemaphoreType.DMA,
          pltpu.SemaphoreType.DMA,
          pltpu.SemaphoreType.DMA,
      ],
      compiler_params=pltpu.CompilerParams(use_tc_tiling_on_sc=True, needs_layout_passes=False),
  )


def _mlp_body(u_ref, i_ref, uid_ref, iid_ref, tu_ref, ti_ref, w1u_ref,
              w1i_ref, b1_ref, w2_ref, b2_ref, wout_ref, bout_ref, out_ref):
  cols = lax.broadcasted_iota(jnp.int32, (1, 64), 1)
  uid = uid_ref[...]
  iid = iid_ref[...]
  oh_u = (uid - TAIL_START == cols).astype(jnp.float32)
  oh_i = (iid - TAIL_START == cols).astype(jnp.float32)
  u = jnp.where(uid >= TAIL_START, oh_u @ tu_ref[...], u_ref[:, :EMB])
  i = jnp.where(iid >= TAIL_START, oh_i @ ti_ref[...], i_ref[:, :EMB])
  h = u @ w1u_ref[...] + i @ w1i_ref[...] + b1_ref[...]
  h = jnp.maximum(h, 0.0)
  h = jnp.maximum(h @ w2_ref[...] + b2_ref[...], 0.0)
  y = jnp.sum(h * wout_ref[...], axis=1, keepdims=True) + bout_ref[...]
  out_ref[...] = y


def _mlp_call(gu, gi, uid2, iid2, tail_u, tail_i, w1u, w1i, b1, w2, b2,
              wout_t, bout, block_b):
  grid = (BATCH // block_b,)
  full = lambda shape: pl.BlockSpec(shape, lambda b: (0,) * len(shape))
  return pl.pallas_call(
      _mlp_body,
      grid=grid,
      in_specs=[
          pl.BlockSpec((block_b, 128), lambda b: (b, 0)),
          pl.BlockSpec((block_b, 128), lambda b: (b, 0)),
          pl.BlockSpec((block_b, 1), lambda b: (b, 0)),
          pl.BlockSpec((block_b, 1), lambda b: (b, 0)),
          full((64, EMB)),
          full((64, EMB)),
          full((EMB, HID)),
          full((EMB, HID)),
          full((1, HID)),
          full((HID, HID)),
          full((1, HID)),
          full((1, HID)),
          full((1, 1)),
      ],
      out_specs=pl.BlockSpec((block_b, 1), lambda b: (b, 0)),
      out_shape=jax.ShapeDtypeStruct((BATCH, 1), jnp.float32),
  )(gu, gi, uid2, iid2, tail_u, tail_i, w1u, w1i, b1, w2, b2, wout_t, bout)


def kernel(userID, ItemID, user_table, item_table, W1, b1, W2, b2, Wout, bout):
  gu, gi = _sc_gather()(userID, ItemID, user_table.T, item_table.T)
  tail_u = user_table[TAIL_START:]
  tail_i = item_table[TAIL_START:]
  y = _mlp_call(gu, gi, userID.reshape(BATCH, 1), ItemID.reshape(BATCH, 1),
                tail_u, tail_i, W1[:EMB], W1[EMB:], b1.reshape(1, HID),
                W2, b2.reshape(1, HID), Wout.reshape(1, HID),
                bout.reshape(1, 1), block_b=2048)
  return y[:, 0]


# speedup vs baseline: 1.1459x; 1.1065x over previous
import functools

import jax
import jax.numpy as jnp
from jax import lax
from jax.experimental import pallas as pl
from jax.experimental.pallas import tpu as pltpu
from jax.experimental.pallas import tpu_sc as plsc

BATCH = 16384
EMB = 32
HID = 64
NROW = 1000000

NC = 2
NS = 16
NW = NC * NS

NT_FULL = 7812
TAIL_START = NT_FULL * 128
CROWS = 640
NCH = -(-245 * 128 // CROWS)
STAGE = 256
CPS = 7
NSUP = 7
SUBCAP = 496
DUMMY = BATCH
NV = BATCH // 16


def _compact(refs, offset, values, m, trash):
  cum = jnp.cumsum(m.astype(jnp.int32))
  pos = jnp.minimum(jnp.where(m, offset + cum - 1, trash), trash)
  for ref, val in zip(refs, values):
    plsc.store_scatter(ref, [pos], val)
  pc = cum[15]
  return offset + pc, pc


def _one_table(idx_hbm, tab_hbm, out_hbm, idx_v, my_idx, my_j, sub_idx,
               sub_j, chunk_v, st_rows, st_j, st_jx, st_pos,
               sem_i, sem_c, sem_o, rs, re):
  iota = lax.iota(jnp.int32, 16)

  pltpu.async_copy(idx_hbm, idx_v, sem_i).wait()

  def scan_ids(v, cnt):
    ids = idx_v[pl.ds(v * 16, 16)]
    m = (ids >= rs) & (ids < re)

    def hit(c):
      c2, _ = _compact([my_idx, my_j], c, [ids, iota + v * 16], m, BATCH + 8)
      return c2

    return lax.cond(jnp.any(m), hit, lambda c: c, cnt)

  cnt = lax.fori_loop(0, NV, scan_ids, 0, unroll=4)
  nv_mine = (cnt + 15) // 16

  dummy_vec = jnp.full((16,), DUMMY, jnp.int32)
  for t in range(STAGE // 16):
    st_j[pl.ds(t * 16, 16)] = dummy_vec

  def chunk_start(c):
    return pl.multiple_of(
        jnp.minimum(rs + c * CROWS, re - CROWS), 128)

  def copy_chunk(c):
    cs = chunk_start(c)
    return pltpu.make_async_copy(
        tab_hbm.at[:, pl.ds(cs, CROWS)], chunk_v.at[c % 2], sem_c)

  def flush(wcnt):
    for t in range(STAGE // 16):
      st_jx[pl.ds(t * 16, 16)] = st_j[pl.ds(t * 16, 16)]
    pltpu.async_copy(st_rows, out_hbm.at[st_jx], sem_o).wait()
    return 0 * wcnt

  def ss_of(s):
    return pl.multiple_of(
        jnp.minimum(rs + s * (CPS * CROWS), re - CPS * CROWS), 128)

  def copy_chunk2(ss, s, k):
    cs = pl.multiple_of(ss + k * CROWS, 128)
    return pltpu.make_async_copy(
        tab_hbm.at[:, pl.ds(cs, CROWS)], chunk_v.at[(s * CPS + k) & 1], sem_c)

  copy_chunk2(ss_of(0), 0, 0).start()

  def do_super(s, wcnt):
    ss = ss_of(s)

    def scan_sub(v, sc):
      ids = my_idx[pl.ds(v * 16, 16)]
      jv = my_j[pl.ds(v * 16, 16)]
      m = (ids >= ss) & (ids < ss + CPS * CROWS) & (iota + v * 16 < cnt)

      def hit(c):
        c2, _ = _compact([sub_idx, sub_j], c, [ids, jv], m, SUBCAP + 8)
        return c2

      return lax.cond(jnp.any(m), hit, lambda x: x, sc)

    scnt = lax.fori_loop(0, nv_mine, scan_sub, 0)

    def chunks_with(src_idx_ref, src_j_ref):
      def run(wcnt, n_entries):
        def do_chunk(k, wc):
          c = s * CPS + k

          @pl.when(c + 1 < NCH)
          def _prefetch():
            s2 = jnp.where(k + 1 < CPS, s, s + 1)
            k2 = jnp.where(k + 1 < CPS, k + 1, 0)
            copy_chunk2(ss_of(s2), s2, k2).start()

          copy_chunk2(ss, s, k).wait()
          cs = pl.multiple_of(ss + k * CROWS, 128)
          par = c & 1
          nv = (n_entries + 15) // 16

          def scan_mine(v, wc):
            ids = src_idx_ref[pl.ds(v * 16, 16)]
            jv = src_j_ref[pl.ds(v * 16, 16)]
            m = (ids >= cs) & (ids < cs + CROWS) & (iota + v * 16 < n_entries)

            def hit(wc):
              wc = lax.cond(wc > STAGE - 16, flush, lambda w: w, wc)
              _, pc = _compact([st_j, st_pos], wc, [jv, ids - cs], m,
                               STAGE + 8)

              def one_entry(e, _):
                pos = st_pos[pl.ds(wc + e, 16)][0]
                psplat = jnp.full((16,), pos, jnp.int32)
                par_splat = jnp.full((16,), par, jnp.int32)
                row_splat = jnp.full((16,), wc + e, jnp.int32)
                g_lo = plsc.load_gather(chunk_v, [par_splat, iota, psplat])
                g_hi = plsc.load_gather(chunk_v,
                                        [par_splat, iota + 16, psplat])
                plsc.store_scatter(st_rows, [row_splat, iota], g_lo)
                plsc.store_scatter(st_rows, [row_splat, iota + 16], g_hi)
                return _

              lax.fori_loop(0, pc, one_entry, 0)
              return wc + pc

            return lax.cond(jnp.any(m), hit, lambda w: w, wc)

          return lax.fori_loop(0, nv, scan_mine, wc)

        return lax.fori_loop(0, CPS, do_chunk, wcnt)

      return run

    return lax.cond(scnt <= SUBCAP,
                    lambda w: chunks_with(sub_idx, sub_j)(w, scnt),
                    lambda w: chunks_with(my_idx, my_j)(w, cnt), wcnt)

  wcnt_final = lax.fori_loop(0, NSUP, do_super, 0)
  flush(wcnt_final)


def _sc_body(uid_hbm, iid_hbm, utab_hbm, itab_hbm, out_u_hbm, out_i_hbm,
             idx_v, my_idx, my_j, sub_idx, sub_j, chunk_v, st_rows, st_j,
             st_jx, st_pos, sem_i, sem_c, sem_o):
  wid = lax.axis_index("s") * NC + lax.axis_index("c")
  ts = (NT_FULL * wid) // NW
  te = (NT_FULL * (wid + 1)) // NW
  rs = pl.multiple_of(ts * 128, 128)
  re = pl.multiple_of(te * 128, 128)
  _one_table(uid_hbm, utab_hbm, out_u_hbm, idx_v, my_idx, my_j, sub_idx,
             sub_j, chunk_v, st_rows, st_j, st_jx, st_pos,
             sem_i, sem_c, sem_o, rs, re)
  _one_table(iid_hbm, itab_hbm, out_i_hbm, idx_v, my_idx, my_j, sub_idx,
             sub_j, chunk_v, st_rows, st_j, st_jx, st_pos,
             sem_i, sem_c, sem_o, rs, re)


@functools.cache
def _sc_gather():
  return pl.kernel(
      _sc_body,
      out_type=(jax.ShapeDtypeStruct((BATCH + 1, 128), jnp.float32),
                jax.ShapeDtypeStruct((BATCH + 1, 128), jnp.float32)),
      mesh=plsc.VectorSubcoreMesh(
          core_axis_name="c", subcore_axis_name="s",
          num_cores=NC, num_subcores=NS),
      scratch_types=[
          pltpu.VMEM((BATCH,), jnp.int32),
          pltpu.VMEM((BATCH + 16,), jnp.int32),
          pltpu.VMEM((BATCH + 16,), jnp.int32),
          pltpu.VMEM((SUBCAP + 24,), jnp.int32),
          pltpu.VMEM((SUBCAP + 24,), jnp.int32),
          pltpu.VMEM((2, EMB, CROWS), jnp.float32),
          pltpu.VMEM((STAGE, 128), jnp.float32),
          pltpu.VMEM((STAGE + 16,), jnp.int32),
          pltpu.VMEM((STAGE,), jnp.int32),
          pltpu.VMEM((STAGE + 16,), jnp.int32),
          pltpu.SemaphoreType.DMA,
          pltpu.SemaphoreType.DMA,
          pltpu.SemaphoreType.DMA,
      ],
      compiler_params=pltpu.CompilerParams(use_tc_tiling_on_sc=True, needs_layout_passes=False),
  )


def _mlp_body(u_ref, i_ref, uid_ref, iid_ref, tu_ref, ti_ref, w1u_ref,
              w1i_ref, b1_ref, w2_ref, b2_ref, wout_ref, bout_ref, out_ref):
  cols = lax.broadcasted_iota(jnp.int32, (1, 64), 1)
  uid = uid_ref[...]
  iid = iid_ref[...]
  oh_u = (uid - TAIL_START == cols).astype(jnp.float32)
  oh_i = (iid - TAIL_START == cols).astype(jnp.float32)
  u = jnp.where(uid >= TAIL_START, oh_u @ tu_ref[...], u_ref[:, :EMB])
  i = jnp.where(iid >= TAIL_START, oh_i @ ti_ref[...], i_ref[:, :EMB])
  h = u @ w1u_ref[...] + i @ w1i_ref[...] + b1_ref[...]
  h = jnp.maximum(h, 0.0)
  h = jnp.maximum(h @ w2_ref[...] + b2_ref[...], 0.0)
  y = jnp.sum(h * wout_ref[...], axis=1, keepdims=True) + bout_ref[...]
  out_ref[...] = y


def _mlp_call(gu, gi, uid2, iid2, tail_u, tail_i, w1u, w1i, b1, w2, b2,
              wout_t, bout, block_b):
  grid = (BATCH // block_b,)
  full = lambda shape: pl.BlockSpec(shape, lambda b: (0,) * len(shape))
  return pl.pallas_call(
      _mlp_body,
      grid=grid,
      in_specs=[
          pl.BlockSpec((block_b, 128), lambda b: (b, 0)),
          pl.BlockSpec((block_b, 128), lambda b: (b, 0)),
          pl.BlockSpec((block_b, 1), lambda b: (b, 0)),
          pl.BlockSpec((block_b, 1), lambda b: (b, 0)),
          full((64, EMB)),
          full((64, EMB)),
          full((EMB, HID)),
          full((EMB, HID)),
          full((1, HID)),
          full((HID, HID)),
          full((1, HID)),
          full((1, HID)),
          full((1, 1)),
      ],
      out_specs=pl.BlockSpec((block_b, 1), lambda b: (b, 0)),
      out_shape=jax.ShapeDtypeStruct((BATCH, 1), jnp.float32),
  )(gu, gi, uid2, iid2, tail_u, tail_i, w1u, w1i, b1, w2, b2, wout_t, bout)


def kernel(userID, ItemID, user_table, item_table, W1, b1, W2, b2, Wout, bout):
  gu, gi = _sc_gather()(userID, ItemID, user_table.T, item_table.T)
  tail_u = user_table[TAIL_START:]
  tail_i = item_table[TAIL_START:]
  y = _mlp_call(gu, gi, userID.reshape(BATCH, 1), ItemID.reshape(BATCH, 1),
                tail_u, tail_i, W1[:EMB], W1[EMB:], b1.reshape(1, HID),
                W2, b2.reshape(1, HID), Wout.reshape(1, HID),
                bout.reshape(1, 1), block_b=2048)
  return y[:, 0]
